# baseline (speedup 1.0000x reference)
import functools

import jax
import jax.numpy as jnp
from jax import lax
from jax.experimental import pallas as pl
from jax.experimental.pallas import tpu as pltpu
from jax.experimental.pallas import tpu_sc as plsc

N = 10000
NPAD = 10240
E = 160000
EPAD = 163840
NC, NS = 2, 16
NW = NC * NS
EW = EPAD // NW
CG = 64
NPAIR_G = EW // (2 * CG)
CS = 128
NCHUNK_S = (EPAD // NS) // CS
CX = 512
NCHUNK_X = (EPAD // NS) // CX
RPT = NPAD // NS
XLEN = 3 * NPAD
XROWS = 256
XREAL = 240

_f32 = jnp.float32
_i32 = jnp.int32


def _gather_body(hl, hp, xl2, xp2, idxcat,
                 hs_o, hd_o, gs_o, gd_o, vd_o,
                 idxb0, idxb1, hba, hbb, vb0, vb1, xlv, xpv,
                 semg, semwh, semv0, semv1, semi0, semi1):
    wid = lax.axis_index("s") * NC + lax.axis_index("c")
    base0 = wid * EW

    pltpu.sync_copy(xl2.at[pl.ds(0, XREAL)], xlv)
    pltpu.sync_copy(xp2.at[pl.ds(0, XREAL)], xpv)

    def cgather(tab, iv):
        return plsc.load_gather(tab, [lax.shift_right_logical(iv, 7),
                                      lax.bitwise_and(iv, 127)])

    outs = (hs_o, hd_o, gs_o, gd_o)
    hb8 = list(hba) + list(hbb)

    def idx_load(m, idxb, semi):
        pltpu.async_copy(idxcat.at[pl.ds(4 * (base0 + m * 2 * CG), 8 * CG)],
                         idxb, semi)

    def drain_hb():
        for t in range(8):
            pltpu.make_async_copy(hb8[t], hs_o.at[pl.ds(0, CG)],
                                  semwh).wait()

    def drain_vb(vb, semv):
        pltpu.make_async_copy(vb, vd_o.at[:, pl.ds(0, 2 * CG)], semv).wait()

    def do_pair(m, idxb, vb, semv, semi, first_hb, first_vb, last):
        b = base0 + m * 2 * CG

        @pl.when(jnp.logical_not(first_hb))
        def _():
            drain_hb()

        @pl.when(jnp.logical_not(first_vb))
        def _():
            drain_vb(vb, semv)

        pltpu.make_async_copy(
            idxcat.at[pl.ds(0, 8 * CG)], idxb, semi).wait()
        descs = []
        for h in range(2):
            o = 4 * CG * h
            descs += [
                pltpu.async_copy(hl.at[idxb.at[pl.ds(o, CG)]],
                                 hb8[4 * h + 0], semg),
                pltpu.async_copy(hp.at[idxb.at[pl.ds(o + CG, CG)]],
                                 hb8[4 * h + 1], semg),
                pltpu.async_copy(hl.at[idxb.at[pl.ds(o + 2 * CG, CG)]],
                                 hb8[4 * h + 2], semg),
                pltpu.async_copy(hl.at[idxb.at[pl.ds(o + 3 * CG, CG)]],
                                 hb8[4 * h + 3], semg),
            ]
        for h in range(2):
            o = 4 * CG * h
            for k in range(CG // 16):
                sl = pl.ds(o + k * 16, 16)
                osl = pl.ds(h * CG + k * 16, 16)
                iv_s = idxb[sl]
                iv_d = idxb[pl.ds(o + CG + k * 16, 16)]
                iv_sl = idxb[pl.ds(o + 2 * CG + k * 16, 16)]
                iv_dl = idxb[pl.ds(o + 3 * CG + k * 16, 16)]
                d2ij = jnp.zeros((16,), _f32)
                d2ik = jnp.zeros((16,), _f32)
                for c in range(3):
                    off = c * NPAD
                    v = cgather(xpv, iv_d + off) - cgather(xlv, iv_s + off)
                    vb[c, osl] = v
                    d2ij = d2ij + v * v
                    w = cgather(xlv, iv_dl + off) - cgather(xlv, iv_sl + off)
                    vb[4 + c, osl] = w
                    d2ik = d2ik + w * w
                vb[3, osl] = d2ij
                vb[7, osl] = d2ik
        for d in descs:
            d.wait()
        for h in range(2):
            bh = b + h * CG
            for t in range(4):
                pltpu.async_copy(hb8[4 * h + t], outs[t].at[pl.ds(bh, CG)],
                                 semwh)
        pltpu.async_copy(vb, vd_o.at[:, pl.ds(b, 2 * CG)], semv)

        @pl.when(jnp.logical_not(last))
        def _():
            idx_load(m + 2, idxb, semi)

    idx_load(0, idxb0, semi0)
    idx_load(1, idxb1, semi1)

    def body(t, carry):
        last = t == NPAIR_G // 2 - 1
        do_pair(2 * t, idxb0, vb0, semv0, semi0, t == 0, t == 0, last)
        do_pair(2 * t + 1, idxb1, vb1, semv1, semi1, False, t == 0, last)
        return carry

    lax.fori_loop(0, NPAIR_G // 2, body, 0, unroll=False)
    drain_hb()
    drain_vb(vb0, semv0)
    drain_vb(vb1, semv1)


@functools.cache
def _gather():
    return pl.kernel(
        _gather_body,
        out_type=[jax.ShapeDtypeStruct((EPAD, 128), _f32)] * 4
        + [jax.ShapeDtypeStruct((8, EPAD), _f32)],
        mesh=plsc.VectorSubcoreMesh(core_axis_name="c", subcore_axis_name="s"),
        compiler_params=pltpu.CompilerParams(needs_layout_passes=False),
        scratch_types=[
            pltpu.VMEM((8 * CG,), _i32),
            pltpu.VMEM((8 * CG,), _i32),
            [pltpu.VMEM((CG, 128), _f32)] * 4,
            [pltpu.VMEM((CG, 128), _f32)] * 4,
            pltpu.VMEM((8, 2 * CG), _f32),
            pltpu.VMEM((8, 2 * CG), _f32),
            pltpu.VMEM((XREAL, 128), _f32),
            pltpu.VMEM((XREAL, 128), _f32),
            pltpu.SemaphoreType.DMA,
            pltpu.SemaphoreType.DMA,
            pltpu.SemaphoreType.DMA,
            pltpu.SemaphoreType.DMA,
            pltpu.SemaphoreType.DMA,
            pltpu.SemaphoreType.DMA,
        ],
    )


def _hscatter_body(hl, hp, src, dst, srcl, m_ij, m_ji, m_ik,
                   hl_o, hp_o,
                   acc_h, idx0, idx1, row0, row1,
                   seml0, seml1, sema0, sema1):
    cid = lax.axis_index("c")
    sid = lax.axis_index("s")
    r0 = sid * RPT
    e0 = sid * (EPAD // NS)

    def loads(idx_hbm, rows_hbm, j, idxb, rowb, seml):
        b = e0 + j * CS
        pltpu.async_copy(idx_hbm.at[pl.ds(b, CS)], idxb, seml)
        pltpu.async_copy(rows_hbm.at[pl.ds(b, CS)], rowb, seml)

    def wait_loads(idxb, rowb, seml):
        pltpu.make_async_copy(src.at[pl.ds(0, CS)], idxb, seml).wait()
        pltpu.make_async_copy(m_ij.at[pl.ds(0, CS)], rowb, seml).wait()

    def fire_add(idxb, rowb, sema):
        pltpu.async_copy(rowb, acc_h.at[idxb], sema, add=True)

    def drain_add(idxb, rowb, sema):
        pltpu.make_async_copy(rowb, acc_h.at[idxb], sema).wait()

    def run(pairs, n_units):
        def unit_src(u):
            a = u % len(pairs)
            j = u // len(pairs)
            return pairs[a][0], pairs[a][1], j

        ih, rh, j = unit_src(0)
        loads(ih, rh, j, idx0, row0, seml0)
        niter = n_units // 2

        def body(m, carry):
            wait_loads(idx0, row0, seml0)
            fire_add(idx0, row0, sema0)

            @pl.when(m > 0)
            def _():
                drain_add(idx1, row1, sema1)

            u1 = 2 * m + 1
            ih1, rh1 = pairs[1 % len(pairs)]
            j1 = u1 // len(pairs)
            loads(ih1, rh1, j1, idx1, row1, seml1)
            wait_loads(idx1, row1, seml1)
            fire_add(idx1, row1, sema1)
            drain_add(idx0, row0, sema0)

            @pl.when(m < niter - 1)
            def _():
                u2 = 2 * m + 2
                ih2, rh2 = pairs[0]
                j2 = u2 // len(pairs)
                loads(ih2, rh2, j2, idx0, row0, seml0)

            return carry

        lax.fori_loop(0, niter, body, 0, unroll=False)
        drain_add(idx1, row1, sema1)

    @pl.when(cid == 0)
    def _():
        pltpu.sync_copy(hl.at[pl.ds(r0, RPT)], acc_h.at[pl.ds(r0, RPT)])
        plsc.subcore_barrier()
        run([(src, m_ij), (srcl, m_ik)], 2 * NCHUNK_S)
        plsc.subcore_barrier()
        pltpu.sync_copy(acc_h.at[pl.ds(r0, RPT)], hl_o.at[pl.ds(r0, RPT)])

    @pl.when(cid == 1)
    def _():
        pltpu.sync_copy(hp.at[pl.ds(r0, RPT)], acc_h.at[pl.ds(r0, RPT)])
        plsc.subcore_barrier()
        run([(dst, m_ji)], NCHUNK_S)
        plsc.subcore_barrier()
        pltpu.sync_copy(acc_h.at[pl.ds(r0, RPT)], hp_o.at[pl.ds(r0, RPT)])


@functools.cache
def _hscatter():
    return pl.kernel(
        _hscatter_body,
        out_type=[
            jax.ShapeDtypeStruct((NPAD, 128), _f32),
            jax.ShapeDtypeStruct((NPAD, 128), _f32),
        ],
        mesh=plsc.VectorSubcoreMesh(core_axis_name="c", subcore_axis_name="s"),
        compiler_params=pltpu.CompilerParams(needs_layout_passes=False),
        scratch_types=[
            pltpu.VMEM_SHARED((NPAD, 128), _f32),
            pltpu.VMEM((CS,), _i32),
            pltpu.VMEM((CS,), _i32),
            pltpu.VMEM((CS, 128), _f32),
            pltpu.VMEM((CS, 128), _f32),
            pltpu.SemaphoreType.DMA,
            pltpu.SemaphoreType.DMA,
            pltpu.SemaphoreType.DMA,
            pltpu.SemaphoreType.DMA,
        ],
    )


def _xscatter_body(xl2, src, srcl, p,
                   xl2_o,
                   acc_sh, accx, idxs0, idxl0, pb0, idxs1, idxl1, pb1,
                   ib0, ib1, redb, baseb, ob, seml0, seml1, sema):
    cid = lax.axis_index("c")
    sid = lax.axis_index("s")

    @pl.when(cid == 1)
    def _():
        e0 = sid * (EPAD // NS)

        def zrow(r, carry):
            for k in range(8):
                accx[r, pl.ds(k * 16, 16)] = jnp.zeros((16,), _f32)
            return carry

        lax.fori_loop(0, XROWS, zrow, 0)
        for k in range(8):
            io = lax.iota(_i32, 16) + k * 16
            ib0[pl.ds(k * 16, 16)] = io
        for k in range(8):
            io = lax.iota(_i32, 16) + k * 16
            ib1[pl.ds(k * 16, 16)] = io + 128

        @pl.when(sid == 0)
        def _():
            pltpu.sync_copy(accx.at[pl.ds(0, XROWS)], acc_sh)

        plsc.subcore_barrier()

        def loads(j, idxs, idxl, pb, seml):
            b = e0 + j * CX
            pltpu.async_copy(src.at[pl.ds(b, CX)], idxs, seml)
            pltpu.async_copy(srcl.at[pl.ds(b, CX)], idxl, seml)
            pltpu.async_copy(p.at[:, pl.ds(b, CX)], pb, seml)

        def wait_loads(idxs, idxl, pb, seml):
            pltpu.make_async_copy(src.at[pl.ds(0, CX)], idxs, seml).wait()
            pltpu.make_async_copy(srcl.at[pl.ds(0, CX)], idxl, seml).wait()
            pltpu.make_async_copy(p.at[:, pl.ds(0, CX)], pb, seml).wait()

        def compute(idxs, idxl, pb):
            for k in range(CX // 16):
                sl = pl.ds(k * 16, 16)
                iv_s = idxs[sl]
                iv_sl = idxl[sl]
                for c in range(3):
                    f1 = iv_s + c * NPAD
                    plsc.addupdate_scatter(
                        accx, [lax.shift_right_logical(f1, 7),
                               lax.bitwise_and(f1, 127)], pb[c, sl])
                    f2 = iv_sl + c * NPAD
                    plsc.addupdate_scatter(
                        accx, [lax.shift_right_logical(f2, 7),
                               lax.bitwise_and(f2, 127)], pb[3 + c, sl])

        loads(0, idxs0, idxl0, pb0, seml0)

        def body(m, carry):
            wait_loads(idxs0, idxl0, pb0, seml0)
            loads(2 * m + 1, idxs1, idxl1, pb1, seml1)
            compute(idxs0, idxl0, pb0)
            wait_loads(idxs1, idxl1, pb1, seml1)

            @pl.when(m < NCHUNK_X // 2 - 1)
            def _():
                loads(2 * m + 2, idxs0, idxl0, pb0, seml0)

            compute(idxs1, idxl1, pb1)
            return carry

        lax.fori_loop(0, NCHUNK_X // 2, body, 0, unroll=False)
        pltpu.sync_copy(accx.at[pl.ds(0, 128)], acc_sh.at[ib0], add=True)
        pltpu.sync_copy(accx.at[pl.ds(128, XROWS - 128)],
                        acc_sh.at[ib1], add=True)
        plsc.subcore_barrier()
        rr = sid * (XROWS // NS)
        pltpu.sync_copy(acc_sh.at[pl.ds(rr, XROWS // NS)], redb)
        pltpu.sync_copy(xl2.at[pl.ds(rr, XROWS // NS)], baseb)
        for r in range(XROWS // NS):
            for k in range(8):
                sl = pl.ds(k * 16, 16)
                ob[r, sl] = redb[r, sl] + baseb[r, sl]
        pltpu.sync_copy(ob, xl2_o.at[pl.ds(rr, XROWS // NS)])


@functools.cache
def _xscatter():
    return pl.kernel(
        _xscatter_body,
        out_type=jax.ShapeDtypeStruct((XROWS, 128), _f32),
        mesh=plsc.VectorSubcoreMesh(core_axis_name="c", subcore_axis_name="s"),
        compiler_params=pltpu.CompilerParams(needs_layout_passes=False),
        scratch_types=[
            pltpu.VMEM_SHARED((XROWS, 128), _f32),
            pltpu.VMEM((XROWS, 128), _f32),
            pltpu.VMEM((CX,), _i32),
            pltpu.VMEM((CX,), _i32),
            pltpu.VMEM((8, CX), _f32),
            pltpu.VMEM((CX,), _i32),
            pltpu.VMEM((CX,), _i32),
            pltpu.VMEM((8, CX), _f32),
            pltpu.VMEM((128,), _i32),
            pltpu.VMEM((128,), _i32),
            pltpu.VMEM((XROWS // NS, 128), _f32),
            pltpu.VMEM((XROWS // NS, 128), _f32),
            pltpu.VMEM((XROWS // NS, 128), _f32),
            pltpu.SemaphoreType.DMA,
            pltpu.SemaphoreType.DMA,
            pltpu.SemaphoreType.DMA,
        ],
    )


def _relu(x):
    return jnp.maximum(x, 0.0)


def _lrelu(x):
    return jnp.where(x >= 0, x, 0.01 * x)


def _mlp_block_body(hs, hd, gs, gd, z, vd,
                    W1h, W1p, W1z, w1d, b1c, W2a, b2a, W2b, b2b,
                    V1a, V1b, v1d, vb1, V2, vb2,
                    X1a, xb1a, X2aT, xb2a, X1b, xb1b, X2bT, xb2b,
                    m_ij_o, m_ji_o, m_ik_o, p_o):
    dot = functools.partial(jnp.dot, preferred_element_type=_f32)
    rt = lambda a, b: lax.dot_general(
        a, b, (((1,), (1,)), ((), ())), preferred_element_type=_f32)

    vdv = vd[...]
    d2ij_r = vdv[3:4, :]
    d2ik_r = vdv[7:8, :]
    u_ij = vdv[0:3, :] / (jnp.sqrt(d2ij_r) + 1e-10)
    u_ik = vdv[4:7, :] / (jnp.sqrt(d2ik_r) + 1e-10)
    ones11 = jnp.ones((1, 1), _f32)
    d_ij = lax.dot_general(jnp.sqrt(d2ij_r), ones11,
                           (((0,), (0,)), ((), ())),
                           preferred_element_type=_f32)
    d_ik = lax.dot_general(jnp.sqrt(d2ik_r), ones11,
                           (((0,), (0,)), ((), ())),
                           preferred_element_type=_f32)

    a = _relu(dot(hs[...], W1h[...]) + dot(hd[...], W1p[...])
              + dot(z[...], W1z[...]) + d_ij * w1d[...] + b1c[...])
    m_ij = _relu(dot(a[:, :128], W2a[...]) + b2a[...])
    m_ji = _relu(dot(a[:, 128:], W2b[...]) + b2b[...])

    c = _relu(dot(gs[...], V1a[...]) + dot(gd[...], V1b[...])
              + d_ik * v1d[...] + vb1[...])
    m_ik = _relu(dot(c, V2[...]) + vb2[...])

    a1g = _lrelu(dot(m_ij, X1a[...]) + xb1a[...])
    a2g = _lrelu(dot(m_ik, X1b[...]) + xb1b[...])
    g1 = _lrelu(rt(X2aT[...], a1g) + xb2a[0, 0])
    g2 = _lrelu(rt(X2bT[...], a2g) + xb2b[0, 0])

    m_ij_o[...] = m_ij
    m_ji_o[...] = m_ji
    m_ik_o[...] = m_ik
    p_o[...] = jnp.concatenate(
        [g1 * u_ij, g2 * u_ik, jnp.zeros_like(u_ij[0:2, :])], axis=0)


def _make_mlp(epad, be):
    grid = (epad // be,)

    def eb(d):
        return pl.BlockSpec((be, d), lambda i: (i, 0))

    def rb():
        return pl.BlockSpec((8, be), lambda i: (0, i))

    def full(shape):
        return pl.BlockSpec(shape, lambda i: tuple(0 for _ in shape))

    in_specs = (
        [eb(128)] * 4 + [eb(64)] + [rb()]
        + [full((128, 256)), full((128, 256)), full((64, 256)),
           full((1, 256)), full((1, 256)),
           full((128, 128)), full((1, 128)), full((128, 128)), full((1, 128)),
           full((128, 128)), full((128, 128)), full((1, 128)), full((1, 128)),
           full((128, 128)), full((1, 128)),
           full((128, 128)), full((1, 128)), full((1, 128)), full((1, 1)),
           full((128, 128)), full((1, 128)), full((1, 128)), full((1, 1))]
    )
    out_specs = [eb(128)] * 3 + [rb()]
    out_shape = (
        [jax.ShapeDtypeStruct((epad, 128), _f32)] * 3
        + [jax.ShapeDtypeStruct((8, epad), _f32)]
    )
    return pl.pallas_call(
        _mlp_block_body,
        grid=grid,
        in_specs=in_specs,
        out_specs=out_specs,
        out_shape=out_shape,
    )


_mlp = _make_mlp(EPAD, 512)


def _layer_weights(params, i):
    p1, p2, pv = params["phi_m1"], params["phi_m2"], params["varphi_m"]
    px1, px2 = params["phi_x1"], params["phi_x2"]
    W1_1, W1_2 = p1["W1"][i], p2["W1"][i]
    W1h = jnp.concatenate([W1_1[:128], W1_2[:128]], axis=1)
    W1p = jnp.concatenate([W1_1[128:256], W1_2[128:256]], axis=1)
    W1z = jnp.concatenate([W1_1[256:320], W1_2[256:320]], axis=1)
    w1d = jnp.concatenate([W1_1[320], W1_2[320]])[None]
    b1c = jnp.concatenate([p1["b1"][i], p2["b1"][i]])[None]
    W2a, b2a = p1["W2"][i], p1["b2"][i][None]
    W2b, b2b = p2["W2"][i], p2["b2"][i][None]
    V1 = pv["W1"][i]
    V1a, V1b, v1d = V1[:128], V1[128:256], V1[256][None]
    vb1 = pv["b1"][i][None]
    V2, vb2 = pv["W2"][i], pv["b2"][i][None]
    X1a, xb1a = px1["W1"][i], px1["b1"][i][None]
    X2aT = px1["W2"][i].T
    xb2a = px1["b2"][i][None]
    X1b, xb1b = px2["W1"][i], px2["b1"][i][None]
    X2bT = px2["W2"][i].T
    xb2b = px2["b2"][i][None]
    return (W1h, W1p, W1z, w1d, b1c, W2a, b2a, W2b, b2b,
            V1a, V1b, v1d, vb1, V2, vb2,
            X1a, xb1a, X2aT, xb2a, X1b, xb1b, X2bT, xb2b)


def _to_x2d(x):
    flat = jnp.pad(x.T, ((0, 0), (0, NPAD - N))).reshape(XLEN)
    return jnp.pad(flat, (0, XROWS * 128 - XLEN)).reshape(XROWS, 128)


def kernel(h_l, h_p, x_l, x_p, edge_index_lp, edge_index_l, z_ij, params):
    src, dst = edge_index_lp[0], edge_index_lp[1]
    src_l, dst_l = edge_index_l[0], edge_index_l[1]

    def pad_e(a):
        return jnp.concatenate([a, jnp.full((EPAD - E,), N, a.dtype)])

    srcp, dstp, srclp, dstlp = pad_e(src), pad_e(dst), pad_e(src_l), pad_e(dst_l)
    hl = jnp.pad(h_l, ((0, NPAD - N), (0, 0)))
    hp = jnp.pad(h_p, ((0, NPAD - N), (0, 0)))
    xl2 = _to_x2d(x_l)
    xp2 = _to_x2d(x_p)
    zp = jnp.pad(z_ij, ((0, EPAD - E), (0, 0)))
    idxcat = (jnp.stack([srcp, dstp, srclp, dstlp])
              .reshape(4, EPAD // CG, CG).transpose(1, 0, 2).reshape(-1))

    for i in range(3):
        wts = _layer_weights(params, i)
        hs, hd, gs, gd, vd = _gather()(hl, hp, xl2, xp2, idxcat)
        m_ij, m_ji, m_ik, p = _mlp(hs, hd, gs, gd, zp, vd, *wts)
        hl, hp = _hscatter()(hl, hp, srcp, dstp, srclp, m_ij, m_ji, m_ik)
        xl2 = _xscatter()(xl2, srcp, srclp, p)

    x_out = xl2.reshape(-1)[:XLEN].reshape(3, NPAD)[:, :N].T
    return (hl[:N], x_out, hp[:N], x_p)


# speedup vs baseline: 1.0474x; 1.0474x over previous
import functools

import jax
import jax.numpy as jnp
from jax import lax
from jax.experimental import pallas as pl
from jax.experimental.pallas import tpu as pltpu
from jax.experimental.pallas import tpu_sc as plsc

N = 10000
NPAD = 10240
E = 160000
EPAD = 163840
NC, NS = 2, 16
NW = NC * NS
EW = EPAD // NW
CG = 128
NCHUNK_G = EW // CG
CS = 128
NCHUNK_S = (EPAD // NS) // CS
CX = 512
NCHUNK_X = (EPAD // NS) // CX
RPT = NPAD // NS
XLEN = 3 * NPAD
XROWS = 256
XREAL = 240

_f32 = jnp.float32
_i32 = jnp.int32


def _gather_body(hl, hp, xl2, xp2, idxcat,
                 hcat_o, vd_o,
                 idxb0, idxb1, gbuf, vdb0, vdb1, xlv, xpv,
                 semg, semv0, semv1, semi0, semi1):
    wid = lax.axis_index("s") * NC + lax.axis_index("c")
    base0 = wid * EW

    pltpu.sync_copy(xl2.at[pl.ds(0, XREAL)], xlv)
    pltpu.sync_copy(xp2.at[pl.ds(0, XREAL)], xpv)

    def cgather(tab, iv):
        return plsc.load_gather(tab, [lax.shift_right_logical(iv, 7),
                                      lax.bitwise_and(iv, 127)])

    def idx_load(c, idxb, semi):
        pltpu.async_copy(idxcat.at[pl.ds(4 * (base0 + c * CG), 4 * CG)],
                         idxb, semi)

    def do_chunk(c, idxb, vdb, semv, semi, first, last):
        b = base0 + c * CG
        pltpu.make_async_copy(idxcat.at[pl.ds(0, 4 * CG)], idxb, semi).wait()
        descs = [
            pltpu.async_copy(hl.at[idxb.at[pl.ds(0, CG)]],
                             gbuf.at[pl.ds(0, CG)], semg),
            pltpu.async_copy(hp.at[idxb.at[pl.ds(CG, CG)]],
                             gbuf.at[pl.ds(CG, CG)], semg),
            pltpu.async_copy(hl.at[idxb.at[pl.ds(2 * CG, CG)]],
                             gbuf.at[pl.ds(2 * CG, CG)], semg),
            pltpu.async_copy(hl.at[idxb.at[pl.ds(3 * CG, CG)]],
                             gbuf.at[pl.ds(3 * CG, CG)], semg),
        ]

        @pl.when(jnp.logical_not(first))
        def _():
            pltpu.make_async_copy(vdb, vd_o.at[:, pl.ds(0, CG)], semv).wait()

        for k in range(CG // 16):
            sl = pl.ds(k * 16, 16)
            iv_s = idxb[sl]
            iv_d = idxb[pl.ds(CG + k * 16, 16)]
            iv_sl = idxb[pl.ds(2 * CG + k * 16, 16)]
            iv_dl = idxb[pl.ds(3 * CG + k * 16, 16)]
            d2ij = jnp.zeros((16,), _f32)
            d2ik = jnp.zeros((16,), _f32)
            for cc in range(3):
                off = cc * NPAD
                v = cgather(xpv, iv_d + off) - cgather(xlv, iv_s + off)
                vdb[cc, sl] = v
                d2ij = d2ij + v * v
                w = cgather(xlv, iv_dl + off) - cgather(xlv, iv_sl + off)
                vdb[4 + cc, sl] = w
                d2ik = d2ik + w * w
            vdb[3, sl] = d2ij
            vdb[7, sl] = d2ik
        pltpu.async_copy(vdb, vd_o.at[:, pl.ds(b, CG)], semv)
        for d in descs:
            d.wait()
        pltpu.sync_copy(gbuf, hcat_o.at[pl.ds(4 * b, 4 * CG)])

        @pl.when(jnp.logical_not(last))
        def _():
            idx_load(c + 2, idxb, semi)

    idx_load(0, idxb0, semi0)
    idx_load(1, idxb1, semi1)

    def body(t, carry):
        last = t == NCHUNK_G // 2 - 1
        do_chunk(2 * t, idxb0, vdb0, semv0, semi0, t == 0, last)
        do_chunk(2 * t + 1, idxb1, vdb1, semv1, semi1, t == 0, last)
        return carry

    lax.fori_loop(0, NCHUNK_G // 2, body, 0, unroll=False)
    pltpu.make_async_copy(vdb0, vd_o.at[:, pl.ds(0, CG)], semv0).wait()
    pltpu.make_async_copy(vdb1, vd_o.at[:, pl.ds(0, CG)], semv1).wait()


@functools.cache
def _gather():
    return pl.kernel(
        _gather_body,
        out_type=[jax.ShapeDtypeStruct((4 * EPAD, 128), _f32),
                  jax.ShapeDtypeStruct((8, EPAD), _f32)],
        mesh=plsc.VectorSubcoreMesh(core_axis_name="c", subcore_axis_name="s"),
        compiler_params=pltpu.CompilerParams(needs_layout_passes=False),
        scratch_types=[
            pltpu.VMEM((4 * CG,), _i32),
            pltpu.VMEM((4 * CG,), _i32),
            pltpu.VMEM((4 * CG, 128), _f32),
            pltpu.VMEM((8, CG), _f32),
            pltpu.VMEM((8, CG), _f32),
            pltpu.VMEM((XREAL, 128), _f32),
            pltpu.VMEM((XREAL, 128), _f32),
            pltpu.SemaphoreType.DMA,
            pltpu.SemaphoreType.DMA,
            pltpu.SemaphoreType.DMA,
            pltpu.SemaphoreType.DMA,
            pltpu.SemaphoreType.DMA,
        ],
    )


def _hscatter_body(hl, hp, src, dst, srcl, m_ij, m_ji, m_ik,
                   hl_o, hp_o,
                   acc_h, idx0, idx1, row0, row1,
                   seml0, seml1, sema0, sema1):
    cid = lax.axis_index("c")
    sid = lax.axis_index("s")
    r0 = sid * RPT
    e0 = sid * (EPAD // NS)

    def loads(idx_hbm, rows_hbm, j, idxb, rowb, seml):
        b = e0 + j * CS
        pltpu.async_copy(idx_hbm.at[pl.ds(b, CS)], idxb, seml)
        pltpu.async_copy(rows_hbm.at[pl.ds(b, CS)], rowb, seml)

    def wait_loads(idxb, rowb, seml):
        pltpu.make_async_copy(src.at[pl.ds(0, CS)], idxb, seml).wait()
        pltpu.make_async_copy(m_ij.at[pl.ds(0, CS)], rowb, seml).wait()

    def fire_add(idxb, rowb, sema):
        pltpu.async_copy(rowb, acc_h.at[idxb], sema, add=True)

    def drain_add(idxb, rowb, sema):
        pltpu.make_async_copy(rowb, acc_h.at[idxb], sema).wait()

    def run(pairs, n_units):
        def unit_src(u):
            a = u % len(pairs)
            j = u // len(pairs)
            return pairs[a][0], pairs[a][1], j

        ih, rh, j = unit_src(0)
        loads(ih, rh, j, idx0, row0, seml0)
        niter = n_units // 2

        def body(m, carry):
            wait_loads(idx0, row0, seml0)
            fire_add(idx0, row0, sema0)

            @pl.when(m > 0)
            def _():
                drain_add(idx1, row1, sema1)

            u1 = 2 * m + 1
            ih1, rh1 = pairs[1 % len(pairs)]
            j1 = u1 // len(pairs)
            loads(ih1, rh1, j1, idx1, row1, seml1)
            wait_loads(idx1, row1, seml1)
            fire_add(idx1, row1, sema1)
            drain_add(idx0, row0, sema0)

            @pl.when(m < niter - 1)
            def _():
                u2 = 2 * m + 2
                ih2, rh2 = pairs[0]
                j2 = u2 // len(pairs)
                loads(ih2, rh2, j2, idx0, row0, seml0)

            return carry

        lax.fori_loop(0, niter, body, 0, unroll=False)
        drain_add(idx1, row1, sema1)

    @pl.when(cid == 0)
    def _():
        pltpu.sync_copy(hl.at[pl.ds(r0, RPT)], acc_h.at[pl.ds(r0, RPT)])
        plsc.subcore_barrier()
        run([(src, m_ij), (srcl, m_ik)], 2 * NCHUNK_S)
        plsc.subcore_barrier()
        pltpu.sync_copy(acc_h.at[pl.ds(r0, RPT)], hl_o.at[pl.ds(r0, RPT)])

    @pl.when(cid == 1)
    def _():
        pltpu.sync_copy(hp.at[pl.ds(r0, RPT)], acc_h.at[pl.ds(r0, RPT)])
        plsc.subcore_barrier()
        run([(dst, m_ji)], NCHUNK_S)
        plsc.subcore_barrier()
        pltpu.sync_copy(acc_h.at[pl.ds(r0, RPT)], hp_o.at[pl.ds(r0, RPT)])


@functools.cache
def _hscatter():
    return pl.kernel(
        _hscatter_body,
        out_type=[
            jax.ShapeDtypeStruct((NPAD, 128), _f32),
            jax.ShapeDtypeStruct((NPAD, 128), _f32),
        ],
        mesh=plsc.VectorSubcoreMesh(core_axis_name="c", subcore_axis_name="s"),
        compiler_params=pltpu.CompilerParams(needs_layout_passes=False),
        scratch_types=[
            pltpu.VMEM_SHARED((NPAD, 128), _f32),
            pltpu.VMEM((CS,), _i32),
            pltpu.VMEM((CS,), _i32),
            pltpu.VMEM((CS, 128), _f32),
            pltpu.VMEM((CS, 128), _f32),
            pltpu.SemaphoreType.DMA,
            pltpu.SemaphoreType.DMA,
            pltpu.SemaphoreType.DMA,
            pltpu.SemaphoreType.DMA,
        ],
    )


def _xscatter_body(xl2, src, srcl, p,
                   xl2_o,
                   acc_sh, accx, idxs0, idxl0, pb0, idxs1, idxl1, pb1,
                   ib0, ib1, redb, baseb, ob, seml0, seml1, sema):
    cid = lax.axis_index("c")
    sid = lax.axis_index("s")

    @pl.when(cid == 1)
    def _():
        e0 = sid * (EPAD // NS)

        def zrow(r, carry):
            for k in range(8):
                accx[r, pl.ds(k * 16, 16)] = jnp.zeros((16,), _f32)
            return carry

        lax.fori_loop(0, XROWS, zrow, 0)
        for k in range(8):
            io = lax.iota(_i32, 16) + k * 16
            ib0[pl.ds(k * 16, 16)] = io
        for k in range(8):
            io = lax.iota(_i32, 16) + k * 16
            ib1[pl.ds(k * 16, 16)] = io + 128

        @pl.when(sid == 0)
        def _():
            pltpu.sync_copy(accx.at[pl.ds(0, XROWS)], acc_sh)

        plsc.subcore_barrier()

        def loads(j, idxs, idxl, pb, seml):
            b = e0 + j * CX
            pltpu.async_copy(src.at[pl.ds(b, CX)], idxs, seml)
            pltpu.async_copy(srcl.at[pl.ds(b, CX)], idxl, seml)
            pltpu.async_copy(p.at[:, pl.ds(b, CX)], pb, seml)

        def wait_loads(idxs, idxl, pb, seml):
            pltpu.make_async_copy(src.at[pl.ds(0, CX)], idxs, seml).wait()
            pltpu.make_async_copy(srcl.at[pl.ds(0, CX)], idxl, seml).wait()
            pltpu.make_async_copy(p.at[:, pl.ds(0, CX)], pb, seml).wait()

        def compute(idxs, idxl, pb):
            for k in range(CX // 16):
                sl = pl.ds(k * 16, 16)
                iv_s = idxs[sl]
                iv_sl = idxl[sl]
                for c in range(3):
                    f1 = iv_s + c * NPAD
                    plsc.addupdate_scatter(
                        accx, [lax.shift_right_logical(f1, 7),
                               lax.bitwise_and(f1, 127)], pb[c, sl])
                    f2 = iv_sl + c * NPAD
                    plsc.addupdate_scatter(
                        accx, [lax.shift_right_logical(f2, 7),
                               lax.bitwise_and(f2, 127)], pb[3 + c, sl])

        loads(0, idxs0, idxl0, pb0, seml0)

        def body(m, carry):
            wait_loads(idxs0, idxl0, pb0, seml0)
            loads(2 * m + 1, idxs1, idxl1, pb1, seml1)
            compute(idxs0, idxl0, pb0)
            wait_loads(idxs1, idxl1, pb1, seml1)

            @pl.when(m < NCHUNK_X // 2 - 1)
            def _():
                loads(2 * m + 2, idxs0, idxl0, pb0, seml0)

            compute(idxs1, idxl1, pb1)
            return carry

        lax.fori_loop(0, NCHUNK_X // 2, body, 0, unroll=False)
        pltpu.sync_copy(accx.at[pl.ds(0, 128)], acc_sh.at[ib0], add=True)
        pltpu.sync_copy(accx.at[pl.ds(128, XROWS - 128)],
                        acc_sh.at[ib1], add=True)
        plsc.subcore_barrier()
        rr = sid * (XROWS // NS)
        pltpu.sync_copy(acc_sh.at[pl.ds(rr, XROWS // NS)], redb)
        pltpu.sync_copy(xl2.at[pl.ds(rr, XROWS // NS)], baseb)
        for r in range(XROWS // NS):
            for k in range(8):
                sl = pl.ds(k * 16, 16)
                ob[r, sl] = redb[r, sl] + baseb[r, sl]
        pltpu.sync_copy(ob, xl2_o.at[pl.ds(rr, XROWS // NS)])


@functools.cache
def _xscatter():
    return pl.kernel(
        _xscatter_body,
        out_type=jax.ShapeDtypeStruct((XROWS, 128), _f32),
        mesh=plsc.VectorSubcoreMesh(core_axis_name="c", subcore_axis_name="s"),
        compiler_params=pltpu.CompilerParams(needs_layout_passes=False),
        scratch_types=[
            pltpu.VMEM_SHARED((XROWS, 128), _f32),
            pltpu.VMEM((XROWS, 128), _f32),
            pltpu.VMEM((CX,), _i32),
            pltpu.VMEM((CX,), _i32),
            pltpu.VMEM((8, CX), _f32),
            pltpu.VMEM((CX,), _i32),
            pltpu.VMEM((CX,), _i32),
            pltpu.VMEM((8, CX), _f32),
            pltpu.VMEM((128,), _i32),
            pltpu.VMEM((128,), _i32),
            pltpu.VMEM((XROWS // NS, 128), _f32),
            pltpu.VMEM((XROWS // NS, 128), _f32),
            pltpu.VMEM((XROWS // NS, 128), _f32),
            pltpu.SemaphoreType.DMA,
            pltpu.SemaphoreType.DMA,
            pltpu.SemaphoreType.DMA,
        ],
    )


def _relu(x):
    return jnp.maximum(x, 0.0)


def _lrelu(x):
    return jnp.where(x >= 0, x, 0.01 * x)


def _mlp_block_body(hcat, z, vd,
                    W1h, W1p, W1z, w1d, b1c, W2a, b2a, W2b, b2b,
                    V1a, V1b, v1d, vb1, V2, vb2,
                    X1a, xb1a, X2aT, xb2a, X1b, xb1b, X2bT, xb2b,
                    m_ij_o, m_ji_o, m_ik_o, p_o):
    dot = functools.partial(jnp.dot, preferred_element_type=_f32)
    rt = lambda a, b: lax.dot_general(
        a, b, (((1,), (1,)), ((), ())), preferred_element_type=_f32)

    hc = hcat[...]
    nch = hc.shape[0] // (4 * CG)

    def deint(t):
        return jnp.concatenate(
            [hc[q * 4 * CG + t * CG:q * 4 * CG + (t + 1) * CG]
             for q in range(nch)], axis=0)

    hs, hd, gs, gd = deint(0), deint(1), deint(2), deint(3)
    vdv = vd[...]
    d2ij_r = vdv[3:4, :]
    d2ik_r = vdv[7:8, :]
    u_ij = vdv[0:3, :] / (jnp.sqrt(d2ij_r) + 1e-10)
    u_ik = vdv[4:7, :] / (jnp.sqrt(d2ik_r) + 1e-10)
    ones11 = jnp.ones((1, 1), _f32)
    d_ij = lax.dot_general(jnp.sqrt(d2ij_r), ones11,
                           (((0,), (0,)), ((), ())),
                           preferred_element_type=_f32)
    d_ik = lax.dot_general(jnp.sqrt(d2ik_r), ones11,
                           (((0,), (0,)), ((), ())),
                           preferred_element_type=_f32)

    a = _relu(dot(hs, W1h[...]) + dot(hd, W1p[...])
              + dot(z[...], W1z[...]) + d_ij * w1d[...] + b1c[...])
    m_ij = _relu(dot(a[:, :128], W2a[...]) + b2a[...])
    m_ji = _relu(dot(a[:, 128:], W2b[...]) + b2b[...])

    c = _relu(dot(gs, V1a[...]) + dot(gd, V1b[...])
              + d_ik * v1d[...] + vb1[...])
    m_ik = _relu(dot(c, V2[...]) + vb2[...])

    a1g = _lrelu(dot(m_ij, X1a[...]) + xb1a[...])
    a2g = _lrelu(dot(m_ik, X1b[...]) + xb1b[...])
    g1 = _lrelu(rt(X2aT[...], a1g) + xb2a[0, 0])
    g2 = _lrelu(rt(X2bT[...], a2g) + xb2b[0, 0])

    m_ij_o[...] = m_ij
    m_ji_o[...] = m_ji
    m_ik_o[...] = m_ik
    p_o[...] = jnp.concatenate(
        [g1 * u_ij, g2 * u_ik, jnp.zeros_like(u_ij[0:2, :])], axis=0)


def _make_mlp(epad, be):
    grid = (epad // be,)

    def eb(d):
        return pl.BlockSpec((be, d), lambda i: (i, 0))

    def rb():
        return pl.BlockSpec((8, be), lambda i: (0, i))

    def full(shape):
        return pl.BlockSpec(shape, lambda i: tuple(0 for _ in shape))

    in_specs = (
        [pl.BlockSpec((4 * be, 128), lambda i: (i, 0))] + [eb(64)] + [rb()]
        + [full((128, 256)), full((128, 256)), full((64, 256)),
           full((1, 256)), full((1, 256)),
           full((128, 128)), full((1, 128)), full((128, 128)), full((1, 128)),
           full((128, 128)), full((128, 128)), full((1, 128)), full((1, 128)),
           full((128, 128)), full((1, 128)),
           full((128, 128)), full((1, 128)), full((1, 128)), full((1, 1)),
           full((128, 128)), full((1, 128)), full((1, 128)), full((1, 1))]
    )
    out_specs = [eb(128)] * 3 + [rb()]
    out_shape = (
        [jax.ShapeDtypeStruct((epad, 128), _f32)] * 3
        + [jax.ShapeDtypeStruct((8, epad), _f32)]
    )
    return pl.pallas_call(
        _mlp_block_body,
        grid=grid,
        in_specs=in_specs,
        out_specs=out_specs,
        out_shape=out_shape,
    )


_mlp = _make_mlp(EPAD, 512)


def _layer_weights(params, i):
    p1, p2, pv = params["phi_m1"], params["phi_m2"], params["varphi_m"]
    px1, px2 = params["phi_x1"], params["phi_x2"]
    W1_1, W1_2 = p1["W1"][i], p2["W1"][i]
    W1h = jnp.concatenate([W1_1[:128], W1_2[:128]], axis=1)
    W1p = jnp.concatenate([W1_1[128:256], W1_2[128:256]], axis=1)
    W1z = jnp.concatenate([W1_1[256:320], W1_2[256:320]], axis=1)
    w1d = jnp.concatenate([W1_1[320], W1_2[320]])[None]
    b1c = jnp.concatenate([p1["b1"][i], p2["b1"][i]])[None]
    W2a, b2a = p1["W2"][i], p1["b2"][i][None]
    W2b, b2b = p2["W2"][i], p2["b2"][i][None]
    V1 = pv["W1"][i]
    V1a, V1b, v1d = V1[:128], V1[128:256], V1[256][None]
    vb1 = pv["b1"][i][None]
    V2, vb2 = pv["W2"][i], pv["b2"][i][None]
    X1a, xb1a = px1["W1"][i], px1["b1"][i][None]
    X2aT = px1["W2"][i].T
    xb2a = px1["b2"][i][None]
    X1b, xb1b = px2["W1"][i], px2["b1"][i][None]
    X2bT = px2["W2"][i].T
    xb2b = px2["b2"][i][None]
    return (W1h, W1p, W1z, w1d, b1c, W2a, b2a, W2b, b2b,
            V1a, V1b, v1d, vb1, V2, vb2,
            X1a, xb1a, X2aT, xb2a, X1b, xb1b, X2bT, xb2b)


def _to_x2d(x):
    flat = jnp.pad(x.T, ((0, 0), (0, NPAD - N))).reshape(XLEN)
    return jnp.pad(flat, (0, XROWS * 128 - XLEN)).reshape(XROWS, 128)


def kernel(h_l, h_p, x_l, x_p, edge_index_lp, edge_index_l, z_ij, params):
    src, dst = edge_index_lp[0], edge_index_lp[1]
    src_l, dst_l = edge_index_l[0], edge_index_l[1]

    def pad_e(a):
        return jnp.concatenate([a, jnp.full((EPAD - E,), N, a.dtype)])

    srcp, dstp, srclp, dstlp = pad_e(src), pad_e(dst), pad_e(src_l), pad_e(dst_l)
    hl = jnp.pad(h_l, ((0, NPAD - N), (0, 0)))
    hp = jnp.pad(h_p, ((0, NPAD - N), (0, 0)))
    xl2 = _to_x2d(x_l)
    xp2 = _to_x2d(x_p)
    zp = jnp.pad(z_ij, ((0, EPAD - E), (0, 0)))
    idxcat = (jnp.stack([srcp, dstp, srclp, dstlp])
              .reshape(4, EPAD // CG, CG).transpose(1, 0, 2).reshape(-1))

    for i in range(3):
        wts = _layer_weights(params, i)
        hcat, vd = _gather()(hl, hp, xl2, xp2, idxcat)
        m_ij, m_ji, m_ik, p = _mlp(hcat, zp, vd, *wts)
        hl, hp = _hscatter()(hl, hp, srcp, dstp, srclp, m_ij, m_ji, m_ik)
        xl2 = _xscatter()(xl2, srcp, srclp, p)

    x_out = xl2.reshape(-1)[:XLEN].reshape(3, NPAD)[:, :N].T
    return (hl[:N], x_out, hp[:N], x_p)
